# baseline (speedup 1.0000x reference)
import functools

import jax
import jax.numpy as jnp
from jax import lax
from jax.experimental import pallas as pl
from jax.experimental.pallas import tpu as pltpu
from jax.experimental.pallas import tpu_sc as plsc

_VOCAB = 100000
_D = 512
_L = 200
_B = 1024
_N = _B * _L
_NC = 2
_NS = 16
_NW = _NC * _NS
_ROWS_PER_W = _N // _NW
_LC = 40
_NCHUNK = _L // _LC
_BATCH_PER_W = _B // _NW
_NPAIR = _BATCH_PER_W // 2
_IDXROWS_PER_W = _NPAIR * _NCHUNK
_LANES = 16
_NBUF = 4
_LAG = 2
_SLOTS = _BATCH_PER_W + _LAG + 2


def _pos_encoding():
    even_i = jnp.arange(0, _D, 2).astype(jnp.float32)
    denom = jnp.power(10000.0, even_i / _D)
    position = jnp.arange(_L).reshape(_L, 1).astype(jnp.float32)
    even_pe = jnp.sin(position / denom)
    odd_pe = jnp.cos(position / denom)
    return jnp.stack([even_pe, odd_pe], axis=2).reshape(_L, _D)


def _sc_embed(table, tokens_pairs, pos):
    mesh = plsc.VectorSubcoreMesh(core_axis_name="c", subcore_axis_name="s")

    @functools.partial(
        pl.kernel,
        mesh=mesh,
        out_type=jax.ShapeDtypeStruct((_N, _D), jnp.float32),
        scratch_types=[
            pltpu.VMEM((_IDXROWS_PER_W, 2 * _LC), jnp.int32),
            pltpu.VMEM((_LC, _D), jnp.float32),
            pltpu.VMEM((2 * _LC, _D), jnp.float32),
            pltpu.VMEM((2 * _LC, _D), jnp.float32),
        ]
        + [pltpu.SemaphoreType.DMA for _ in range(6)],
    )
    def k(table_hbm, tok_hbm, pos_hbm, out_hbm,
          idx_v, pos_v, p0, p1, gs0, gs1, w0, w1, w2, w3):
        pbufs = (p0, p1)
        gsems = (gs0, gs1)
        wsems = (w0, w1, w2, w3)
        wid = lax.axis_index("s") * _NC + lax.axis_index("c")
        base = wid * _ROWS_PER_W
        pltpu.sync_copy(
            tok_hbm.at[pl.ds(wid * _IDXROWS_PER_W, _IDXROWS_PER_W)], idx_v
        )

        def slot_view(s):
            return pbufs[s // 2].at[pl.ds((s % 2) * _LC, _LC)]

        def wait_write(s):
            pltpu.make_async_copy(
                slot_view(s), out_hbm.at[pl.ds(0, _LC)], wsems[s]
            ).wait()

        def wait_pair_gather(ph):
            pltpu.make_async_copy(
                table_hbm.at[pl.ds(0, 2 * _LC)], pbufs[ph], gsems[ph]
            ).wait()

        def chunk_body(c, carry):
            pltpu.sync_copy(pos_hbm.at[pl.ds(c * _LC, _LC)], pos_v)

            def slot_group(j, carry2):
                for b in range(_NBUF):
                    i = j * _NBUF + b
                    g = i
                    u = i - _LAG
                    bu = (b + _NBUF - _LAG) % _NBUF

                    if b % 2 == 0:
                        @pl.when(g < _BATCH_PER_W)
                        def _issue():
                            @pl.when(c * _BATCH_PER_W + g >= _NBUF)
                            def _drain():
                                wait_write(b)
                                wait_write(b + 1)

                            row = (g // 2) * _NCHUNK + c
                            pltpu.async_copy(
                                table_hbm.at[idx_v.at[row]],
                                pbufs[b // 2], gsems[b // 2],
                            )

                    @pl.when(jnp.logical_and(u >= 0, u < _BATCH_PER_W))
                    def _consume():
                        if bu % 2 == 0:
                            wait_pair_gather(bu // 2)
                        rv = slot_view(bu)

                        def add_body(r, carry3):
                            for kk in range(_D // _LANES):
                                sl = pl.ds(kk * _LANES, _LANES)
                                plsc.addupdate(rv.at[r, sl], pos_v[r, sl])
                            return carry3

                        lax.fori_loop(0, _LC, add_body, 0)
                        out_off = base + u * _L + c * _LC
                        pltpu.async_copy(
                            rv, out_hbm.at[pl.ds(out_off, _LC)], wsems[bu]
                        )
                return carry2

            lax.fori_loop(0, _SLOTS // _NBUF, slot_group, 0)
            return carry

        lax.fori_loop(0, _NCHUNK, chunk_body, 0)
        for s in range(_NBUF):
            wait_write(s)

    return k(table, tokens_pairs, pos)


def kernel(tokens, table):
    pos = _pos_encoding()
    tokens_pairs = (
        tokens.astype(jnp.int32)
        .reshape(_NW, _NPAIR, 2, _NCHUNK, _LC)
        .transpose(0, 1, 3, 2, 4)
        .reshape(_NW * _IDXROWS_PER_W, 2 * _LC)
    )
    out = _sc_embed(table, tokens_pairs, pos)
    return out.reshape(_B, _L, _D)
